# baseline (speedup 1.0000x reference)
import dataclasses
import functools
import jax
import jax.numpy as jnp
from jax import lax
from jax.experimental import pallas as pl
from jax.experimental.pallas import tpu as pltpu
from jax.experimental.pallas import tpu_sc as plsc

N = 10000
E = 160000
T = 3
IN = 256
OUT = 256
HEADS = 4
SHEAD = 2
HID = 128
NEG = 0.2

NPAD = 10240
EPRIME = E + N
EPAD = 170496


NC = 2
NS = 16
NW = NC * NS
CH = EPAD // NW
NTAB = NPAD + 64
EGROUPS = CH // 16

_SC_MESH = dict(core_axis_name="c", subcore_axis_name="s")


def _sc_params():
    cp = pltpu.CompilerParams()
    if "needs_layout_passes" in pltpu.CompilerParams.__dataclass_fields__:
        cp = dataclasses.replace(cp, needs_layout_passes=False)
    return cp


def _scex_body(src_hbm, dst_hbm, tsrc_hbm, tdst_hbm, m_hbm, ex_hbm,
               src_v, dst_v, tsrc_v, tdst_v, ex_v, m_v):
    c = lax.axis_index("c")
    s = lax.axis_index("s")
    wid = c * NS + s
    base = wid * CH
    pltpu.sync_copy(src_hbm.at[pl.ds(base, CH)], src_v)
    pltpu.sync_copy(dst_hbm.at[pl.ds(base, CH)], dst_v)
    pltpu.sync_copy(tsrc_hbm, tsrc_v)
    pltpu.sync_copy(tdst_hbm, tdst_v)
    pltpu.sync_copy(m_hbm, m_v)

    s0 = src_v[pl.ds(0, 16)]
    d0 = dst_v[pl.ds(0, 16)]
    mhs = [m_v[pl.ds(h * 16, 16)] for h in range(HEADS)]

    def _group(g, carry):
        s16, d16 = carry
        o = g * 16
        nxt = jnp.minimum(o + 16, CH - 16)
        ns16 = src_v[pl.ds(nxt, 16)]
        nd16 = dst_v[pl.ds(nxt, 16)]
        for h in range(HEADS):
            a = (plsc.load_gather(tsrc_v, [s16 * HEADS + h]) +
                 plsc.load_gather(tdst_v, [d16 * HEADS + h]))
            a = jnp.maximum(a, 0.0) + NEG * jnp.minimum(a, 0.0)
            ex_v[pl.ds(h * CH + o, 16)] = jnp.exp(a - mhs[h])
        return (ns16, nd16)

    lax.fori_loop(0, EGROUPS, _group, (s0, d0))

    for h in range(HEADS):
        pltpu.sync_copy(ex_v.at[pl.ds(h * CH, CH)],
                        ex_hbm.at[pl.ds(h * EPAD + base, CH)])


def _scex_call(src, dst, tsrc, tdst, m16):
    k = pl.kernel(
        _scex_body,
        out_type=jax.ShapeDtypeStruct((HEADS * EPAD,), jnp.float32),
        mesh=plsc.VectorSubcoreMesh(**_SC_MESH),
        compiler_params=_sc_params(),
        scratch_types=[
            pltpu.VMEM((CH,), jnp.int32),
            pltpu.VMEM((CH,), jnp.int32),
            pltpu.VMEM((NTAB * HEADS,), jnp.float32),
            pltpu.VMEM((NTAB * HEADS,), jnp.float32),
            pltpu.VMEM((HEADS * CH,), jnp.float32),
            pltpu.VMEM((HEADS * 16,), jnp.float32),
        ],
    )
    return k(src, dst, tsrc, tdst, m16)


NB = NPAD // 64
NLOC = CH + (NB + 1) * 16
RECW = NLOC * 8
NPASS = NB // NW


def _rd(ref, idx):
    return ref[pl.ds(idx, 16)][0]


def _asm16(read_fn, dtype):
    lane = lax.iota(jnp.int32, 16)
    v = jnp.full((16,), read_fn(0), dtype)
    for i in range(1, 16):
        v = jnp.where(lane == i, jnp.full((16,), read_fn(i), dtype), v)
    return v


def _scbin_body(src_hbm, dst_hbm, ex_hbm, rec_hbm, offs_hbm,
                src_v, dst_v, ex_v, rec_v, btmp_v, off_s):
    c = lax.axis_index("c")
    s = lax.axis_index("s")
    wid = c * NS + s
    base = wid * CH
    pltpu.sync_copy(src_hbm.at[pl.ds(base, CH)], src_v)
    pltpu.sync_copy(dst_hbm.at[pl.ds(base, CH)], dst_v)
    for h in range(HEADS):
        pltpu.sync_copy(ex_hbm.at[pl.ds(h * EPAD + base, CH)],
                        ex_v.at[pl.ds(h * CH, CH)])

    lane = lax.iota(jnp.int32, 16)
    patt = jnp.where(lax.rem(lane, 8) == 1, float(NPAD), 0.0)

    @pl.loop(0, RECW, step=16)
    def _(i):
        rec_v[pl.ds(i, 16)] = patt

    @pl.loop(0, NB + 1)
    def _(b):
        off_s[b] = 0

    @pl.loop(0, CH, step=16)
    def _(o):
        b16 = lax.shift_right_logical(dst_v[pl.ds(o, 16)], 6)
        for i in range(16):
            b = b16[i]
            off_s[b] = off_s[b] + 1

    def _pref(b, run):
        off_s[176 + b] = run
        off_s[352 + b] = run
        nb = lax.shift_left(lax.shift_right_logical(off_s[b] + 15, 4), 4)
        return run + nb
    lax.fori_loop(0, NB + 1, _pref, jnp.int32(0))

    @pl.loop(0, CH, step=16)
    def _(o):
        d16 = dst_v[pl.ds(o, 16)]
        s16f = src_v[pl.ds(o, 16)].astype(jnp.float32)
        d16f = d16.astype(jnp.float32)
        b16 = lax.shift_right_logical(d16, 6)

        def _claim(i):
            b = b16[i]
            p = off_s[352 + b]
            off_s[352 + b] = p + 1
            return p
        p_vec = _asm16(_claim, jnp.int32)
        r = p_vec * 8
        plsc.store_scatter(rec_v, [r], s16f)
        plsc.store_scatter(rec_v, [r + 1], d16f)
        for h in range(HEADS):
            plsc.store_scatter(rec_v, [r + 2 + h], ex_v[pl.ds(h * CH + o, 16)])

    for g in range(11):
        btmp_v[pl.ds(g * 16, 16)] = _asm16(
            lambda i: off_s[176 + g * 16 + i], jnp.int32)
    pltpu.sync_copy(rec_v, rec_hbm.at[pl.ds(wid * RECW, RECW)])
    pltpu.sync_copy(btmp_v.at[pl.ds(0, 176)],
                    offs_hbm.at[pl.ds(wid * 176, 176)])


def _scbin_call(src, dst, ex_flat):
    k = pl.kernel(
        _scbin_body,
        out_type=(jax.ShapeDtypeStruct((NW * RECW,), jnp.float32),
                  jax.ShapeDtypeStruct((NW * 176,), jnp.int32)),
        mesh=plsc.VectorSubcoreMesh(**_SC_MESH),
        compiler_params=_sc_params(),
        scratch_types=[
            pltpu.VMEM((CH,), jnp.int32),
            pltpu.VMEM((CH,), jnp.int32),
            pltpu.VMEM((HEADS * CH,), jnp.float32),
            pltpu.VMEM((RECW,), jnp.float32),
            pltpu.VMEM((176,), jnp.int32),
            pltpu.SMEM((544,), jnp.int32),
        ],
    )
    return k(src, dst, ex_flat)


WG = 8
WE = WG * 16


def _scacc_body(rec_hbm, offs_hbm, x_hbm, acc_hbm, den_hbm,
                offs_v, recw_v, idx_v, ln_v, exb_v, xrows_v, acc_v, den_v,
                den_s, sem, gsem):
    cc = lax.axis_index("c")
    s = lax.axis_index("s")
    wid = cc * NS + s
    pltpu.sync_copy(offs_hbm, offs_v.at[pl.ds(0, NW * 176)])

    @pl.loop(0, WE // 16)
    def _(i):
        idx_v[pl.ds(i * 16, 16)] = jnp.zeros((16,), jnp.int32)

    iota16 = lax.iota(jnp.int32, 16)

    @pl.loop(0, NPASS)
    def _(p):
        b = p * NW + wid

        @pl.loop(0, 64 * HEADS * IN, step=16)
        def _(i):
            acc_v[pl.ds(i, 16)] = jnp.zeros((16,), jnp.float32)

        @pl.loop(0, 64 * HEADS)
        def _(i):
            den_s[i] = jnp.float32(0.0)

        @pl.loop(0, NW)
        def _(c):
            st = _rd(offs_v, c * 176 + b)
            en = _rd(offs_v, c * 176 + b + 1)
            glen = lax.shift_right_logical(en - st, 4)
            rbase = c * RECW + st * 8
            nwin = lax.shift_right_logical(glen + (WG - 1), 3)

            @pl.loop(0, nwin)
            def _(wi):
                gcnt = jnp.minimum(glen - wi * WG, WG)

                @pl.loop(0, gcnt)
                def _(j):
                    pltpu.async_copy(
                        rec_hbm.at[pl.ds(rbase + (wi * WG + j) * 128, 128)],
                        recw_v.at[pl.ds(j * 128, 128)], sem)

                @pl.loop(0, gcnt)
                def _(j):
                    pltpu.make_async_copy(
                        rec_hbm.at[pl.ds(0, 128)],
                        recw_v.at[pl.ds(0, 128)], sem).wait()

                @pl.loop(0, gcnt)
                def _(j):
                    gb = j * 128
                    srcf = plsc.load_gather(recw_v, [iota16 * 8 + gb])
                    dstf = plsc.load_gather(recw_v, [iota16 * 8 + (gb + 1)])
                    idx_v[pl.ds(j * 16, 16)] = srcf.astype(jnp.int32)
                    ln = dstf.astype(jnp.int32) - b * 64
                    ln = jnp.clip(ln, 0, 63)
                    ln_v[pl.ds(j * 16, 16)] = ln
                    for h in range(HEADS):
                        exf = plsc.load_gather(recw_v,
                                               [iota16 * 8 + (gb + 2 + h)])
                        exb_v[pl.ds(h * WE + j * 16, 16)] = exf

                pltpu.async_copy(x_hbm.at[idx_v], xrows_v, gsem)
                pltpu.make_async_copy(x_hbm.at[idx_v], xrows_v, gsem).wait()

                @pl.loop(0, gcnt)
                def _(j):
                    jb = j * 16
                    ln16 = ln_v[pl.ds(jb, 16)]
                    ex16 = [exb_v[pl.ds(h * WE + jb, 16)]
                            for h in range(HEADS)]
                    for i in range(16):
                        ln = ln16[i]
                        ab = ln * (HEADS * IN)
                        e = jb + i
                        exs = []
                        for h in range(HEADS):
                            ex_s = ex16[h][i]
                            exs.append(jnp.full((16,), ex_s, jnp.float32))
                            dh = ln * HEADS + h
                            den_s[dh] = den_s[dh] + ex_s

                        @pl.loop(0, 4)
                        def _(cq):
                            for cs in range(4):
                                co = cq * 64 + cs * 16
                                rv = xrows_v[e, pl.ds(co, 16)]
                                for h in range(HEADS):
                                    plsc.addupdate(
                                        acc_v.at[pl.ds(ab + h * IN + co,
                                                       16)],
                                        rv * exs[h])

        for g in range(16):
            den_v[pl.ds(g * 16, 16)] = _asm16(
                lambda i: den_s[g * 16 + i], jnp.float32)
        pltpu.sync_copy(acc_v, acc_hbm.at[pl.ds(b * 64 * HEADS * IN,
                                                64 * HEADS * IN)])
        pltpu.sync_copy(den_v, den_hbm.at[pl.ds(b * 64 * HEADS,
                                                64 * HEADS)])


def _scacc_call(rec, offs, xpad):
    k = pl.kernel(
        _scacc_body,
        out_type=(jax.ShapeDtypeStruct((NPAD * HEADS * IN,), jnp.float32),
                  jax.ShapeDtypeStruct((NPAD * HEADS,), jnp.float32)),
        mesh=plsc.VectorSubcoreMesh(**_SC_MESH),
        compiler_params=_sc_params(),
        scratch_types=[
            pltpu.VMEM((NW * 176 + 16,), jnp.int32),
            pltpu.VMEM((WG * 128,), jnp.float32),
            pltpu.VMEM((WE,), jnp.int32),
            pltpu.VMEM((WE + 16,), jnp.int32),
            pltpu.VMEM((HEADS * WE + 16,), jnp.float32),
            pltpu.VMEM((WE, IN), jnp.float32),
            pltpu.VMEM((64 * HEADS * IN,), jnp.float32),
            pltpu.VMEM((64 * HEADS,), jnp.float32),
            pltpu.SMEM((64 * HEADS,), jnp.float32),
            pltpu.SemaphoreType.DMA,
            pltpu.SemaphoreType.DMA,
        ],
    )
    return k(rec, offs, xpad)


def _alpha_kernel(x_ref, w_ref, a_ref, m_ref):
    a = jnp.dot(x_ref[...], w_ref[...], preferred_element_type=jnp.float32)
    a_ref[...] = a
    m_ref[...] = jnp.max(a, axis=0, keepdims=True)


def _alpha_call(xpad, wcat):
    return pl.pallas_call(
        _alpha_kernel,
        out_shape=(jax.ShapeDtypeStruct((NPAD, 128), jnp.float32),
                   jax.ShapeDtypeStruct((1, 128), jnp.float32)),
    )(xpad, wcat)


def _posts_kernel(acc_ref, den_ref, wp_ref, bias_ref, gwu_ref, gwx_ref,
                  gb_ref, x_ref, emb_ref):
    inv = 1.0 / (den_ref[...] + 1e-16)
    divisor = jnp.concatenate(
        [jnp.broadcast_to(inv[:, h:h + 1], (inv.shape[0], IN))
         for h in range(HEADS)], axis=1)
    u = jnp.dot(acc_ref[...] * divisor, wp_ref[...],
                preferred_element_type=jnp.float32) * (1.0 / HEADS)
    u = u + bias_ref[...]
    g = u @ gwu_ref[...] + x_ref[...] @ gwx_ref[...] + gb_ref[...]
    g = jax.nn.sigmoid(g)
    emb_ref[...] = jnp.tanh(u) * g + x_ref[...] * (1.0 - g)


def _posts_call(acc, den_pad, wp, bias, gwu, gwx, gb, xpad):
    BN = 2048
    grid = (NPAD // BN,)
    return pl.pallas_call(
        _posts_kernel,
        grid=grid,
        in_specs=[
            pl.BlockSpec((BN, HEADS * IN), lambda i: (i, 0)),
            pl.BlockSpec((BN, 128), lambda i: (i, 0)),
            pl.BlockSpec((HEADS * IN, OUT), lambda i: (0, 0)),
            pl.BlockSpec((1, OUT), lambda i: (0, 0)),
            pl.BlockSpec((OUT, IN), lambda i: (0, 0)),
            pl.BlockSpec((IN, IN), lambda i: (0, 0)),
            pl.BlockSpec((1, IN), lambda i: (0, 0)),
            pl.BlockSpec((BN, IN), lambda i: (i, 0)),
        ],
        out_specs=pl.BlockSpec((BN, OUT), lambda i: (i, 0)),
        out_shape=jax.ShapeDtypeStruct((NPAD, OUT), jnp.float32),
    )(acc, den_pad, wp, bias, gwu, gwx, gb, xpad)


def _sem_kernel(e0_ref, e1_ref, e2_ref, w1_ref, b1_ref, w2_ref, wbar_ref,
                s_acc):
    i = pl.program_id(0)
    nblocks = pl.num_programs(0)

    @pl.when(i == 0)
    def _():
        for k in range(SHEAD):
            for t in range(T):
                s_acc[k, t] = jnp.float32(0.0)

    base = i * e0_ref.shape[0]
    row = base + lax.broadcasted_iota(jnp.int32, (e0_ref.shape[0], 1), 0)
    mask = (row < N).astype(jnp.float32)
    embs = (e0_ref[...], e1_ref[...], e2_ref[...])
    for k in range(SHEAD):
        w1 = w1_ref[k]
        b1 = b1_ref[k]
        w2 = w2_ref[k]
        for t in range(T):
            h = jnp.tanh(jnp.dot(embs[t], w1,
                                 preferred_element_type=jnp.float32) + b1)
            v = jnp.dot(h, w2.T, preferred_element_type=jnp.float32)
            s_acc[k, t] += jnp.sum(v[:, :1] * mask)

    @pl.when(i == nblocks - 1)
    def _():
        wbar = [jnp.float32(0.0)] * T
        for k in range(SHEAD):
            sk = [s_acc[k, t] * (1.0 / N) for t in range(T)]
            mx = jnp.maximum(jnp.maximum(sk[0], sk[1]), sk[2])
            ek = [jnp.exp(s - mx) for s in sk]
            tot = ek[0] + ek[1] + ek[2]
            for t in range(T):
                wbar[t] = wbar[t] + ek[t] / tot
        for t in range(T):
            wbar_ref[0, t] = wbar[t] * (1.0 / SHEAD)


def _sem_call(e0, e1, e2, w1, b1, w2):
    BN = 2048
    return pl.pallas_call(
        _sem_kernel,
        grid=(NPAD // BN,),
        in_specs=[
            pl.BlockSpec((BN, OUT), lambda i: (i, 0)),
            pl.BlockSpec((BN, OUT), lambda i: (i, 0)),
            pl.BlockSpec((BN, OUT), lambda i: (i, 0)),
            pl.BlockSpec((SHEAD, OUT, HID), lambda i: (0, 0, 0)),
            pl.BlockSpec((SHEAD, 1, HID), lambda i: (0, 0, 0)),
            pl.BlockSpec((SHEAD, 1, HID), lambda i: (0, 0, 0)),
        ],
        out_specs=pl.BlockSpec((1, 8), lambda i: (0, 0),
                               memory_space=pltpu.SMEM),
        out_shape=jax.ShapeDtypeStruct((1, 8), jnp.float32),
        scratch_shapes=[pltpu.SMEM((SHEAD, 8), jnp.float32)],
    )(e0, e1, e2, w1, b1, w2)


def _comb_kernel(wbar_ref, e0_ref, e1_ref, e2_ref, out_ref):
    out_ref[...] = (wbar_ref[0, 0] * e0_ref[...] +
                    wbar_ref[0, 1] * e1_ref[...] +
                    wbar_ref[0, 2] * e2_ref[...])


def _comb_call(wbar, e0, e1, e2):
    BN = 2048
    return pl.pallas_call(
        _comb_kernel,
        grid=(NPAD // BN,),
        in_specs=[
            pl.BlockSpec((1, 8), lambda i: (0, 0), memory_space=pltpu.SMEM),
            pl.BlockSpec((BN, OUT), lambda i: (i, 0)),
            pl.BlockSpec((BN, OUT), lambda i: (i, 0)),
            pl.BlockSpec((BN, OUT), lambda i: (i, 0)),
        ],
        out_specs=pl.BlockSpec((BN, OUT), lambda i: (i, 0)),
        out_shape=jax.ShapeDtypeStruct((NPAD, OUT), jnp.float32),
    )(wbar, e0, e1, e2)


def kernel(features, edge_index_list, gat_W, gat_att_src, gat_att_dst,
           gat_bias, gate_W, gate_b, sa_W1, sa_b1, sa_W2):
    x = features
    xpad = jnp.zeros((NPAD, IN), jnp.float32).at[:N].set(x)

    W4 = gat_W.reshape(T, IN, HEADS, OUT)
    watt_src = jnp.einsum('tiho,tho->tih', W4, gat_att_src)
    watt_dst = jnp.einsum('tiho,tho->tih', W4, gat_att_dst)
    wcat = jnp.zeros((IN, 128), jnp.float32)
    wcat = wcat.at[:, :T * HEADS].set(watt_src.transpose(1, 0, 2)
                                      .reshape(IN, T * HEADS))
    wcat = wcat.at[:, 12:12 + T * HEADS].set(watt_dst.transpose(1, 0, 2)
                                             .reshape(IN, T * HEADS))
    Wp = W4.transpose(0, 2, 1, 3).reshape(T, HEADS * IN, OUT)
    gwu = gate_W[:OUT]
    gwx = gate_W[OUT:]

    A, M = _alpha_call(xpad, wcat)

    loop = jnp.arange(N, dtype=jnp.int32)
    pad_e = EPAD - EPRIME
    embs = []
    for t in range(T):
        src = jnp.concatenate([edge_index_list[t, 0].astype(jnp.int32), loop,
                               jnp.zeros((pad_e,), jnp.int32)])
        dst = jnp.concatenate([edge_index_list[t, 1].astype(jnp.int32), loop,
                               jnp.full((pad_e,), NPAD, jnp.int32)])
        asrc_t = A[:, t * HEADS:(t + 1) * HEADS]
        adst_t = A[:, 12 + t * HEADS:12 + (t + 1) * HEADS]
        m = M[0, t * HEADS:(t + 1) * HEADS] + \
            M[0, 12 + t * HEADS:12 + (t + 1) * HEADS]
        tsrc = jnp.zeros((NTAB, HEADS), jnp.float32).at[:NPAD].set(asrc_t)
        tdst = jnp.zeros((NTAB, HEADS), jnp.float32).at[:NPAD].set(adst_t)
        m16 = jnp.repeat(m, 16)
        ex = _scex_call(src, dst, tsrc.reshape(-1), tdst.reshape(-1), m16)
        rec, offs = _scbin_call(src, dst, ex)
        accU, den = _scacc_call(rec, offs, xpad)
        accU = accU.reshape(NPAD, HEADS * IN)
        den_pad = jnp.zeros((NPAD, 128), jnp.float32).at[:, :HEADS].set(
            den.reshape(NPAD, HEADS))
        emb = _posts_call(accU, den_pad, Wp[t], gat_bias[t][None], gwu, gwx,
                          gate_b[None], xpad)
        embs.append(emb)

    wbar = _sem_call(embs[0], embs[1], embs[2], sa_W1, sa_b1[:, None, :],
                     sa_W2.transpose(0, 2, 1))
    outp = _comb_call(wbar, embs[0], embs[1], embs[2])
    return outp[:N]


# speedup vs baseline: 6.7163x; 6.7163x over previous
import dataclasses
import functools
import jax
import jax.numpy as jnp
from jax import lax
from jax.experimental import pallas as pl
from jax.experimental.pallas import tpu as pltpu
from jax.experimental.pallas import tpu_sc as plsc

N = 10000
E = 160000
T = 3
IN = 256
OUT = 256
HEADS = 4
SHEAD = 2
HID = 128
NEG = 0.2

NPAD = 10240
EPRIME = E + N
EPAD = 170496


NC = 2
NS = 16
NW = NC * NS
CH = EPAD // NW
NTAB = NPAD + 64
EGROUPS = CH // 16

_SC_MESH = dict(core_axis_name="c", subcore_axis_name="s")


def _sc_params():
    cp = pltpu.CompilerParams()
    if "needs_layout_passes" in pltpu.CompilerParams.__dataclass_fields__:
        cp = dataclasses.replace(cp, needs_layout_passes=False)
    return cp


def _scex_body(src_hbm, dst_hbm, tsrc_hbm, tdst_hbm, m_hbm, ex_hbm,
               src_v, dst_v, tsrc_v, tdst_v, ex_v, m_v):
    c = lax.axis_index("c")
    s = lax.axis_index("s")
    wid = c * NS + s
    base = wid * CH
    pltpu.sync_copy(src_hbm.at[pl.ds(base, CH)], src_v)
    pltpu.sync_copy(dst_hbm.at[pl.ds(base, CH)], dst_v)
    pltpu.sync_copy(tsrc_hbm, tsrc_v)
    pltpu.sync_copy(tdst_hbm, tdst_v)
    pltpu.sync_copy(m_hbm, m_v)

    s0 = src_v[pl.ds(0, 16)]
    d0 = dst_v[pl.ds(0, 16)]
    mhs = [m_v[pl.ds(h * 16, 16)] for h in range(HEADS)]

    def _group(g, carry):
        s16, d16 = carry
        o = g * 16
        nxt = jnp.minimum(o + 16, CH - 16)
        ns16 = src_v[pl.ds(nxt, 16)]
        nd16 = dst_v[pl.ds(nxt, 16)]
        for h in range(HEADS):
            a = (plsc.load_gather(tsrc_v, [s16 * HEADS + h]) +
                 plsc.load_gather(tdst_v, [d16 * HEADS + h]))
            a = jnp.maximum(a, 0.0) + NEG * jnp.minimum(a, 0.0)
            ex_v[pl.ds(h * CH + o, 16)] = jnp.exp(a - mhs[h])
        return (ns16, nd16)

    lax.fori_loop(0, EGROUPS, _group, (s0, d0))

    for h in range(HEADS):
        pltpu.sync_copy(ex_v.at[pl.ds(h * CH, CH)],
                        ex_hbm.at[pl.ds(h * EPAD + base, CH)])


def _scex_call(src, dst, tsrc, tdst, m16):
    k = pl.kernel(
        _scex_body,
        out_type=jax.ShapeDtypeStruct((HEADS * EPAD,), jnp.float32),
        mesh=plsc.VectorSubcoreMesh(**_SC_MESH),
        compiler_params=_sc_params(),
        scratch_types=[
            pltpu.VMEM((CH,), jnp.int32),
            pltpu.VMEM((CH,), jnp.int32),
            pltpu.VMEM((NTAB * HEADS,), jnp.float32),
            pltpu.VMEM((NTAB * HEADS,), jnp.float32),
            pltpu.VMEM((HEADS * CH,), jnp.float32),
            pltpu.VMEM((HEADS * 16,), jnp.float32),
        ],
    )
    return k(src, dst, tsrc, tdst, m16)


NB = NPAD // 64
NLOC = CH + (NB + 1) * 16
RECW = NLOC * 8
NPASS = NB // NW


def _rd(ref, idx):
    return ref[pl.ds(idx, 16)][0]


def _asm16(read_fn, dtype):
    lane = lax.iota(jnp.int32, 16)
    v = jnp.full((16,), read_fn(0), dtype)
    for i in range(1, 16):
        v = jnp.where(lane == i, jnp.full((16,), read_fn(i), dtype), v)
    return v


def _scbin_body(src_hbm, dst_hbm, ex_hbm, rec_hbm, offs_hbm,
                src_v, dst_v, ex_v, rec_v, btmp_v, off_s):
    c = lax.axis_index("c")
    s = lax.axis_index("s")
    wid = c * NS + s
    base = wid * CH
    pltpu.sync_copy(src_hbm.at[pl.ds(base, CH)], src_v)
    pltpu.sync_copy(dst_hbm.at[pl.ds(base, CH)], dst_v)
    for h in range(HEADS):
        pltpu.sync_copy(ex_hbm.at[pl.ds(h * EPAD + base, CH)],
                        ex_v.at[pl.ds(h * CH, CH)])

    lane = lax.iota(jnp.int32, 16)
    patt = jnp.where(lax.rem(lane, 8) == 1, float(NPAD), 0.0)

    @pl.loop(0, RECW, step=16)
    def _(i):
        rec_v[pl.ds(i, 16)] = patt

    @pl.loop(0, NB + 1)
    def _(b):
        off_s[b] = 0

    @pl.loop(0, CH, step=16)
    def _(o):
        b16 = lax.shift_right_logical(dst_v[pl.ds(o, 16)], 6)
        for i in range(16):
            b = b16[i]
            off_s[b] = off_s[b] + 1

    def _pref(b, run):
        off_s[176 + b] = run
        off_s[352 + b] = run
        nb = lax.shift_left(lax.shift_right_logical(off_s[b] + 15, 4), 4)
        return run + nb
    lax.fori_loop(0, NB + 1, _pref, jnp.int32(0))

    @pl.loop(0, CH, step=16)
    def _(o):
        d16 = dst_v[pl.ds(o, 16)]
        s16f = src_v[pl.ds(o, 16)].astype(jnp.float32)
        d16f = d16.astype(jnp.float32)
        b16 = lax.shift_right_logical(d16, 6)

        def _claim(i):
            b = b16[i]
            p = off_s[352 + b]
            off_s[352 + b] = p + 1
            return p
        p_vec = _asm16(_claim, jnp.int32)
        r = p_vec * 8
        plsc.store_scatter(rec_v, [r], s16f)
        plsc.store_scatter(rec_v, [r + 1], d16f)
        for h in range(HEADS):
            plsc.store_scatter(rec_v, [r + 2 + h], ex_v[pl.ds(h * CH + o, 16)])

    for g in range(11):
        btmp_v[pl.ds(g * 16, 16)] = _asm16(
            lambda i: off_s[176 + g * 16 + i], jnp.int32)
    pltpu.sync_copy(rec_v, rec_hbm.at[pl.ds(wid * RECW, RECW)])
    pltpu.sync_copy(btmp_v.at[pl.ds(0, 176)],
                    offs_hbm.at[pl.ds(wid * 176, 176)])


def _scbin_call(src, dst, ex_flat):
    k = pl.kernel(
        _scbin_body,
        out_type=(jax.ShapeDtypeStruct((NW * RECW,), jnp.float32),
                  jax.ShapeDtypeStruct((NW * 176,), jnp.int32)),
        mesh=plsc.VectorSubcoreMesh(**_SC_MESH),
        compiler_params=_sc_params(),
        scratch_types=[
            pltpu.VMEM((CH,), jnp.int32),
            pltpu.VMEM((CH,), jnp.int32),
            pltpu.VMEM((HEADS * CH,), jnp.float32),
            pltpu.VMEM((RECW,), jnp.float32),
            pltpu.VMEM((176,), jnp.int32),
            pltpu.SMEM((544,), jnp.int32),
        ],
    )
    return k(src, dst, ex_flat)


WG = 8
WE = WG * 16


def _scacc_body(rec_hbm, offs_hbm, x_hbm, acc_hbm, den_hbm,
                offs_v, recw_v, idx_v, ln_v, exb_v, xrows_v, acc_v, den_v,
                den_s, sem, gsem):
    cc = lax.axis_index("c")
    s = lax.axis_index("s")
    wid = cc * NS + s
    pltpu.sync_copy(offs_hbm, offs_v.at[pl.ds(0, NW * 176)])

    @pl.loop(0, WE // 16)
    def _(i):
        idx_v[pl.ds(i * 16, 16)] = jnp.zeros((16,), jnp.int32)

    iota16 = lax.iota(jnp.int32, 16)

    @pl.loop(0, NPASS)
    def _(p):
        b = p * NW + wid

        @pl.loop(0, 64 * HEADS * IN, step=16)
        def _(i):
            acc_v[pl.ds(i, 16)] = jnp.zeros((16,), jnp.float32)

        @pl.loop(0, 64 * HEADS)
        def _(i):
            den_s[i] = jnp.float32(0.0)

        @pl.loop(0, NW)
        def _(c):
            st = _rd(offs_v, c * 176 + b)
            en = _rd(offs_v, c * 176 + b + 1)
            glen = lax.shift_right_logical(en - st, 4)
            rbase = c * RECW + st * 8
            nwin = lax.shift_right_logical(glen + (WG - 1), 3)

            @pl.loop(0, nwin)
            def _(wi):
                gcnt = jnp.minimum(glen - wi * WG, WG)

                @pl.loop(0, gcnt)
                def _(j):
                    pltpu.async_copy(
                        rec_hbm.at[pl.ds(rbase + (wi * WG + j) * 128, 128)],
                        recw_v.at[pl.ds(j * 128, 128)], sem)

                @pl.loop(0, gcnt)
                def _(j):
                    pltpu.make_async_copy(
                        rec_hbm.at[pl.ds(0, 128)],
                        recw_v.at[pl.ds(0, 128)], sem).wait()

                @pl.loop(0, gcnt)
                def _(j):
                    gb = j * 128
                    srcf = plsc.load_gather(recw_v, [iota16 * 8 + gb])
                    dstf = plsc.load_gather(recw_v, [iota16 * 8 + (gb + 1)])
                    idx_v[pl.ds(j * 16, 16)] = srcf.astype(jnp.int32)
                    ln = dstf.astype(jnp.int32) - b * 64
                    ln = jnp.clip(ln, 0, 63)
                    ln_v[pl.ds(j * 16, 16)] = ln
                    for h in range(HEADS):
                        exf = plsc.load_gather(recw_v,
                                               [iota16 * 8 + (gb + 2 + h)])
                        exb_v[pl.ds(h * WE + j * 16, 16)] = exf

                @pl.loop(0, gcnt)
                def _(j):
                    pltpu.async_copy(x_hbm.at[idx_v.at[pl.ds(j * 16, 16)]],
                                     xrows_v.at[pl.ds(j * 16, 16)], gsem)

                @pl.loop(0, gcnt)
                def _(j):
                    pltpu.make_async_copy(
                        x_hbm.at[idx_v.at[pl.ds(0, 16)]],
                        xrows_v.at[pl.ds(0, 16)], gsem).wait()

                @pl.loop(0, gcnt)
                def _(j):
                    jb = j * 16
                    ln16 = ln_v[pl.ds(jb, 16)]
                    ex16 = [exb_v[pl.ds(h * WE + jb, 16)]
                            for h in range(HEADS)]
                    for i in range(16):
                        ln = ln16[i]
                        ab = ln * (HEADS * IN)
                        e = jb + i
                        exs = []
                        for h in range(HEADS):
                            ex_s = ex16[h][i]
                            exs.append(jnp.full((16,), ex_s, jnp.float32))
                            dh = ln * HEADS + h
                            den_s[dh] = den_s[dh] + ex_s

                        @pl.loop(0, 4)
                        def _(cq):
                            for cs in range(4):
                                co = cq * 64 + cs * 16
                                rv = xrows_v[e, pl.ds(co, 16)]
                                for h in range(HEADS):
                                    plsc.addupdate(
                                        acc_v.at[pl.ds(ab + h * IN + co,
                                                       16)],
                                        rv * exs[h])

        for g in range(16):
            den_v[pl.ds(g * 16, 16)] = _asm16(
                lambda i: den_s[g * 16 + i], jnp.float32)
        pltpu.sync_copy(acc_v, acc_hbm.at[pl.ds(b * 64 * HEADS * IN,
                                                64 * HEADS * IN)])
        pltpu.sync_copy(den_v, den_hbm.at[pl.ds(b * 64 * HEADS,
                                                64 * HEADS)])


def _scacc_call(rec, offs, xpad):
    k = pl.kernel(
        _scacc_body,
        out_type=(jax.ShapeDtypeStruct((NPAD * HEADS * IN,), jnp.float32),
                  jax.ShapeDtypeStruct((NPAD * HEADS,), jnp.float32)),
        mesh=plsc.VectorSubcoreMesh(**_SC_MESH),
        compiler_params=_sc_params(),
        scratch_types=[
            pltpu.VMEM((NW * 176 + 16,), jnp.int32),
            pltpu.VMEM((WG * 128,), jnp.float32),
            pltpu.VMEM((WE,), jnp.int32),
            pltpu.VMEM((WE + 16,), jnp.int32),
            pltpu.VMEM((HEADS * WE + 16,), jnp.float32),
            pltpu.VMEM((WE, IN), jnp.float32),
            pltpu.VMEM((64 * HEADS * IN,), jnp.float32),
            pltpu.VMEM((64 * HEADS,), jnp.float32),
            pltpu.SMEM((64 * HEADS,), jnp.float32),
            pltpu.SemaphoreType.DMA,
            pltpu.SemaphoreType.DMA,
        ],
    )
    return k(rec, offs, xpad)


def _alpha_kernel(x_ref, w_ref, a_ref, m_ref):
    a = jnp.dot(x_ref[...], w_ref[...], preferred_element_type=jnp.float32)
    a_ref[...] = a
    m_ref[...] = jnp.max(a, axis=0, keepdims=True)


def _alpha_call(xpad, wcat):
    return pl.pallas_call(
        _alpha_kernel,
        out_shape=(jax.ShapeDtypeStruct((NPAD, 128), jnp.float32),
                   jax.ShapeDtypeStruct((1, 128), jnp.float32)),
    )(xpad, wcat)


def _posts_kernel(acc_ref, den_ref, wp_ref, bias_ref, gwu_ref, gwx_ref,
                  gb_ref, x_ref, emb_ref):
    inv = 1.0 / (den_ref[...] + 1e-16)
    divisor = jnp.concatenate(
        [jnp.broadcast_to(inv[:, h:h + 1], (inv.shape[0], IN))
         for h in range(HEADS)], axis=1)
    u = jnp.dot(acc_ref[...] * divisor, wp_ref[...],
                preferred_element_type=jnp.float32) * (1.0 / HEADS)
    u = u + bias_ref[...]
    g = u @ gwu_ref[...] + x_ref[...] @ gwx_ref[...] + gb_ref[...]
    g = jax.nn.sigmoid(g)
    emb_ref[...] = jnp.tanh(u) * g + x_ref[...] * (1.0 - g)


def _posts_call(acc, den_pad, wp, bias, gwu, gwx, gb, xpad):
    BN = 2048
    grid = (NPAD // BN,)
    return pl.pallas_call(
        _posts_kernel,
        grid=grid,
        in_specs=[
            pl.BlockSpec((BN, HEADS * IN), lambda i: (i, 0)),
            pl.BlockSpec((BN, 128), lambda i: (i, 0)),
            pl.BlockSpec((HEADS * IN, OUT), lambda i: (0, 0)),
            pl.BlockSpec((1, OUT), lambda i: (0, 0)),
            pl.BlockSpec((OUT, IN), lambda i: (0, 0)),
            pl.BlockSpec((IN, IN), lambda i: (0, 0)),
            pl.BlockSpec((1, IN), lambda i: (0, 0)),
            pl.BlockSpec((BN, IN), lambda i: (i, 0)),
        ],
        out_specs=pl.BlockSpec((BN, OUT), lambda i: (i, 0)),
        out_shape=jax.ShapeDtypeStruct((NPAD, OUT), jnp.float32),
    )(acc, den_pad, wp, bias, gwu, gwx, gb, xpad)


def _sem_kernel(e0_ref, e1_ref, e2_ref, w1_ref, b1_ref, w2_ref, wbar_ref,
                s_acc):
    i = pl.program_id(0)
    nblocks = pl.num_programs(0)

    @pl.when(i == 0)
    def _():
        for k in range(SHEAD):
            for t in range(T):
                s_acc[k, t] = jnp.float32(0.0)

    base = i * e0_ref.shape[0]
    row = base + lax.broadcasted_iota(jnp.int32, (e0_ref.shape[0], 1), 0)
    mask = (row < N).astype(jnp.float32)
    embs = (e0_ref[...], e1_ref[...], e2_ref[...])
    for k in range(SHEAD):
        w1 = w1_ref[k]
        b1 = b1_ref[k]
        w2 = w2_ref[k]
        for t in range(T):
            h = jnp.tanh(jnp.dot(embs[t], w1,
                                 preferred_element_type=jnp.float32) + b1)
            v = jnp.dot(h, w2.T, preferred_element_type=jnp.float32)
            s_acc[k, t] += jnp.sum(v[:, :1] * mask)

    @pl.when(i == nblocks - 1)
    def _():
        wbar = [jnp.float32(0.0)] * T
        for k in range(SHEAD):
            sk = [s_acc[k, t] * (1.0 / N) for t in range(T)]
            mx = jnp.maximum(jnp.maximum(sk[0], sk[1]), sk[2])
            ek = [jnp.exp(s - mx) for s in sk]
            tot = ek[0] + ek[1] + ek[2]
            for t in range(T):
                wbar[t] = wbar[t] + ek[t] / tot
        for t in range(T):
            wbar_ref[0, t] = wbar[t] * (1.0 / SHEAD)


def _sem_call(e0, e1, e2, w1, b1, w2):
    BN = 2048
    return pl.pallas_call(
        _sem_kernel,
        grid=(NPAD // BN,),
        in_specs=[
            pl.BlockSpec((BN, OUT), lambda i: (i, 0)),
            pl.BlockSpec((BN, OUT), lambda i: (i, 0)),
            pl.BlockSpec((BN, OUT), lambda i: (i, 0)),
            pl.BlockSpec((SHEAD, OUT, HID), lambda i: (0, 0, 0)),
            pl.BlockSpec((SHEAD, 1, HID), lambda i: (0, 0, 0)),
            pl.BlockSpec((SHEAD, 1, HID), lambda i: (0, 0, 0)),
        ],
        out_specs=pl.BlockSpec((1, 8), lambda i: (0, 0),
                               memory_space=pltpu.SMEM),
        out_shape=jax.ShapeDtypeStruct((1, 8), jnp.float32),
        scratch_shapes=[pltpu.SMEM((SHEAD, 8), jnp.float32)],
    )(e0, e1, e2, w1, b1, w2)


def _comb_kernel(wbar_ref, e0_ref, e1_ref, e2_ref, out_ref):
    out_ref[...] = (wbar_ref[0, 0] * e0_ref[...] +
                    wbar_ref[0, 1] * e1_ref[...] +
                    wbar_ref[0, 2] * e2_ref[...])


def _comb_call(wbar, e0, e1, e2):
    BN = 2048
    return pl.pallas_call(
        _comb_kernel,
        grid=(NPAD // BN,),
        in_specs=[
            pl.BlockSpec((1, 8), lambda i: (0, 0), memory_space=pltpu.SMEM),
            pl.BlockSpec((BN, OUT), lambda i: (i, 0)),
            pl.BlockSpec((BN, OUT), lambda i: (i, 0)),
            pl.BlockSpec((BN, OUT), lambda i: (i, 0)),
        ],
        out_specs=pl.BlockSpec((BN, OUT), lambda i: (i, 0)),
        out_shape=jax.ShapeDtypeStruct((NPAD, OUT), jnp.float32),
    )(wbar, e0, e1, e2)


def kernel(features, edge_index_list, gat_W, gat_att_src, gat_att_dst,
           gat_bias, gate_W, gate_b, sa_W1, sa_b1, sa_W2):
    x = features
    xpad = jnp.zeros((NPAD, IN), jnp.float32).at[:N].set(x)

    W4 = gat_W.reshape(T, IN, HEADS, OUT)
    watt_src = jnp.einsum('tiho,tho->tih', W4, gat_att_src)
    watt_dst = jnp.einsum('tiho,tho->tih', W4, gat_att_dst)
    wcat = jnp.zeros((IN, 128), jnp.float32)
    wcat = wcat.at[:, :T * HEADS].set(watt_src.transpose(1, 0, 2)
                                      .reshape(IN, T * HEADS))
    wcat = wcat.at[:, 12:12 + T * HEADS].set(watt_dst.transpose(1, 0, 2)
                                             .reshape(IN, T * HEADS))
    Wp = W4.transpose(0, 2, 1, 3).reshape(T, HEADS * IN, OUT)
    gwu = gate_W[:OUT]
    gwx = gate_W[OUT:]

    A, M = _alpha_call(xpad, wcat)

    loop = jnp.arange(N, dtype=jnp.int32)
    pad_e = EPAD - EPRIME
    embs = []
    for t in range(T):
        src = jnp.concatenate([edge_index_list[t, 0].astype(jnp.int32), loop,
                               jnp.zeros((pad_e,), jnp.int32)])
        dst = jnp.concatenate([edge_index_list[t, 1].astype(jnp.int32), loop,
                               jnp.full((pad_e,), NPAD, jnp.int32)])
        asrc_t = A[:, t * HEADS:(t + 1) * HEADS]
        adst_t = A[:, 12 + t * HEADS:12 + (t + 1) * HEADS]
        m = M[0, t * HEADS:(t + 1) * HEADS] + \
            M[0, 12 + t * HEADS:12 + (t + 1) * HEADS]
        tsrc = jnp.zeros((NTAB, HEADS), jnp.float32).at[:NPAD].set(asrc_t)
        tdst = jnp.zeros((NTAB, HEADS), jnp.float32).at[:NPAD].set(adst_t)
        m16 = jnp.repeat(m, 16)
        ex = _scex_call(src, dst, tsrc.reshape(-1), tdst.reshape(-1), m16)
        rec, offs = _scbin_call(src, dst, ex)
        accU, den = _scacc_call(rec, offs, xpad)
        accU = accU.reshape(NPAD, HEADS * IN)
        den_pad = jnp.zeros((NPAD, 128), jnp.float32).at[:, :HEADS].set(
            den.reshape(NPAD, HEADS))
        emb = _posts_call(accU, den_pad, Wp[t], gat_bias[t][None], gwu, gwx,
                          gate_b[None], xpad)
        embs.append(emb)

    wbar = _sem_call(embs[0], embs[1], embs[2], sa_W1, sa_b1[:, None, :],
                     sa_W2.transpose(0, 2, 1))
    outp = _comb_call(wbar, embs[0], embs[1], embs[2])
    return outp[:N]
